# baseline (speedup 1.0000x reference)
import math
from functools import partial

import jax
import jax.numpy as jnp
from jax.experimental import pallas as pl
from jax.experimental.pallas import tpu as pltpu

N, D, H = 2048, 2048, 16
DH = D // H
SCALE = 1.0 / math.sqrt(DH)
EPS = 1e-5

TM = 512
TO = 512
DA = 2 * DH

_DIMS_NT = (((1,), (1,)), ((), ()))
_DIMS_NN = (((1,), (0,)), ((), ()))
_BF = jnp.bfloat16


def _proj_body(x_ref, w_ref, b_ref, g_ref, gt_ref, o_ref, *, mode):
    y = jax.lax.dot_general(
        x_ref[...].astype(_BF), w_ref[...].astype(_BF),
        _DIMS_NT, preferred_element_type=jnp.float32)
    y = y + b_ref[...]
    if mode in ("q", "k"):
        gs = jax.lax.dot_general(
            y * y, g_ref[...], _DIMS_NN, preferred_element_type=jnp.float32)
        norm = jnp.sqrt(gs)
        max_norm = 1.0 - EPS
        scale = jnp.where(norm > max_norm,
                          max_norm / jnp.maximum(norm, 1e-12), 1.0)
        if mode == "q":
            scale = scale * SCALE
        y = y * jax.lax.dot_general(
            scale, gt_ref[...], _DIMS_NN, preferred_element_type=jnp.float32)
        o_ref[...] = y.astype(_BF)
    else:
        yb = y.astype(_BF)
        for h in range(H):
            o_ref[:, h * DA:h * DA + DH] = yb[:, h * DH:(h + 1) * DH]
            o_ref[:, h * DA + DH:(h + 1) * DA] = jnp.ones(
                (y.shape[0], DH), dtype=_BF)


def _proj(x, W, b, g, gt, mode):
    out_d = 2 * D if mode == "v" else D
    return pl.pallas_call(
        partial(_proj_body, mode=mode),
        grid=(N // TM,),
        in_specs=[
            pl.BlockSpec((TM, D), lambda i: (i, 0)),
            pl.BlockSpec((D, D), lambda i: (0, 0)),
            pl.BlockSpec((1, D), lambda i: (0, 0)),
            pl.BlockSpec((D, H), lambda i: (0, 0)),
            pl.BlockSpec((H, D), lambda i: (0, 0)),
        ],
        out_specs=pl.BlockSpec((TM, out_d), lambda i: (i, 0)),
        out_shape=jax.ShapeDtypeStruct((N, out_d), _BF),
    )(x, W, b.reshape(1, D), g, gt)


def _attn_out_body(q_ref, k_ref, v_ref, wo_ref, bo_ref, o_ref, acc_ref):
    for h in range(H):
        sl = slice(h * DH, (h + 1) * DH)
        s = jax.lax.dot_general(
            q_ref[:, sl], k_ref[:, sl], _DIMS_NT,
            preferred_element_type=jnp.float32)
        p = jnp.exp(s.astype(_BF))
        ov = jnp.dot(p, v_ref[:, h * DA:(h + 1) * DA],
                     preferred_element_type=jnp.float32)
        oh = ov[:, :DH] / ov[:, DH:DH + 1]
        acc_ref[:, sl] = oh.astype(_BF)
    o = jax.lax.dot_general(
        acc_ref[...], wo_ref[...].astype(_BF), _DIMS_NT,
        preferred_element_type=jnp.float32)
    o_ref[...] = o + bo_ref[...]


def _attn_out(q, k, v, Wo, bo):
    return pl.pallas_call(
        _attn_out_body,
        grid=(N // TO,),
        in_specs=[
            pl.BlockSpec((TO, D), lambda i: (i, 0)),
            pl.BlockSpec((N, D), lambda i: (0, 0)),
            pl.BlockSpec((N, 2 * D), lambda i: (0, 0)),
            pl.BlockSpec((D, D), lambda i: (0, 0)),
            pl.BlockSpec((1, D), lambda i: (0, 0)),
        ],
        out_specs=pl.BlockSpec((TO, D), lambda i: (i, 0)),
        out_shape=jax.ShapeDtypeStruct((N, D), jnp.float32),
        scratch_shapes=[pltpu.VMEM((TO, D), _BF)],
    )(q, k, v, Wo, bo.reshape(1, D))


def kernel(query, key_, value, Wq, bq, Wk, bk, Wv, bv, Wo, bo):
    g = (jnp.arange(D)[:, None] // DH == jnp.arange(H)[None, :]).astype(jnp.float32)
    gt = g.T
    q = _proj(query.reshape(N, D), Wq, bq, g, gt, mode="q")
    k = _proj(key_.reshape(N, D), Wk, bk, g, gt, mode="k")
    v = _proj(value.reshape(N, D), Wv, bv, g, gt, mode="v")
    out = _attn_out(q, k, v, Wo, bo)
    return out.reshape(1, N, D)


# speedup vs baseline: 1.0384x; 1.0384x over previous
import math
from functools import partial

import jax
import jax.numpy as jnp
from jax.experimental import pallas as pl
from jax.experimental.pallas import tpu as pltpu

N, D, H = 2048, 2048, 16
DH = D // H
SCALE = 1.0 / math.sqrt(DH)
EPS = 1e-5

TM = 512
TO = 256
DA = 2 * DH

_DIMS_NT = (((1,), (1,)), ((), ()))
_DIMS_NN = (((1,), (0,)), ((), ()))
_BF = jnp.bfloat16


def _proj_body(x_ref, w_ref, b_ref, g_ref, gt_ref, o_ref, *, mode):
    y = jax.lax.dot_general(
        x_ref[...].astype(_BF), w_ref[...].astype(_BF),
        _DIMS_NT, preferred_element_type=jnp.float32)
    y = y + b_ref[...]
    if mode in ("q", "k"):
        gs = jax.lax.dot_general(
            y * y, g_ref[...], _DIMS_NN, preferred_element_type=jnp.float32)
        norm = jnp.sqrt(gs)
        max_norm = 1.0 - EPS
        scale = jnp.where(norm > max_norm,
                          max_norm / jnp.maximum(norm, 1e-12), 1.0)
        if mode == "q":
            scale = scale * SCALE
        y = y * jax.lax.dot_general(
            scale, gt_ref[...], _DIMS_NN, preferred_element_type=jnp.float32)
        o_ref[...] = y.astype(_BF)
    else:
        yb = y.astype(_BF)
        for h in range(H):
            o_ref[:, h * DA:h * DA + DH] = yb[:, h * DH:(h + 1) * DH]
            o_ref[:, h * DA + DH:(h + 1) * DA] = jnp.ones(
                (y.shape[0], DH), dtype=_BF)


def _proj(x, W, b, g, gt, mode):
    out_d = 2 * D if mode == "v" else D
    return pl.pallas_call(
        partial(_proj_body, mode=mode),
        grid=(N // TM,),
        in_specs=[
            pl.BlockSpec((TM, D), lambda i: (i, 0)),
            pl.BlockSpec((D, D), lambda i: (0, 0)),
            pl.BlockSpec((1, D), lambda i: (0, 0)),
            pl.BlockSpec((D, H), lambda i: (0, 0)),
            pl.BlockSpec((H, D), lambda i: (0, 0)),
        ],
        out_specs=pl.BlockSpec((TM, out_d), lambda i: (i, 0)),
        out_shape=jax.ShapeDtypeStruct((N, out_d), _BF),
    )(x, W, b.reshape(1, D), g, gt)


def _attn_out_body(q_ref, k_ref, v_ref, wo_ref, bo_ref, o_ref, acc_ref):
    for h in range(H):
        sl = slice(h * DH, (h + 1) * DH)
        s = jax.lax.dot_general(
            q_ref[:, sl], k_ref[:, sl], _DIMS_NT,
            preferred_element_type=jnp.float32)
        p = jnp.exp(s.astype(_BF))
        ov = jnp.dot(p, v_ref[:, h * DA:(h + 1) * DA],
                     preferred_element_type=jnp.float32)
        oh = ov[:, :DH] / ov[:, DH:DH + 1]
        acc_ref[:, sl] = oh.astype(_BF)
    o = jax.lax.dot_general(
        acc_ref[...], wo_ref[...].astype(_BF), _DIMS_NT,
        preferred_element_type=jnp.float32)
    o_ref[...] = o + bo_ref[...]


def _attn_out(q, k, v, Wo, bo):
    return pl.pallas_call(
        _attn_out_body,
        grid=(N // TO,),
        in_specs=[
            pl.BlockSpec((TO, D), lambda i: (i, 0)),
            pl.BlockSpec((N, D), lambda i: (0, 0)),
            pl.BlockSpec((N, 2 * D), lambda i: (0, 0)),
            pl.BlockSpec((D, D), lambda i: (0, 0)),
            pl.BlockSpec((1, D), lambda i: (0, 0)),
        ],
        out_specs=pl.BlockSpec((TO, D), lambda i: (i, 0)),
        out_shape=jax.ShapeDtypeStruct((N, D), jnp.float32),
        scratch_shapes=[pltpu.VMEM((TO, D), _BF)],
    )(q, k, v, Wo, bo.reshape(1, D))


def kernel(query, key_, value, Wq, bq, Wk, bk, Wv, bv, Wo, bo):
    g = (jnp.arange(D)[:, None] // DH == jnp.arange(H)[None, :]).astype(jnp.float32)
    gt = g.T
    q = _proj(query.reshape(N, D), Wq, bq, g, gt, mode="q")
    k = _proj(key_.reshape(N, D), Wk, bk, g, gt, mode="k")
    v = _proj(value.reshape(N, D), Wv, bv, g, gt, mode="v")
    out = _attn_out(q, k, v, Wo, bo)
    return out.reshape(1, N, D)
